# baseline (speedup 1.0000x reference)
import functools

import jax
import jax.numpy as jnp
from jax import lax
from jax.experimental import pallas as pl
from jax.experimental.pallas import tpu as pltpu
from jax.experimental.pallas import tpu_sc as plsc

LTM_SLOTS = 100
VECTOR_DIM = 64
PAD_S = 128
NCHUNK = PAD_S // 16
DECAY_RATE = 0.995
IMPORTANCE_THRESHOLD = 0.45
SIMILARITY_THRESHOLD = 0.85
OLD_WEIGHT = 0.8
NEW_WEIGHT = 0.2
BOOST_FACTOR = 0.5
NEG_BIG = -3.4e38
PAD_STRENGTH = 1e9


def _rsqrt16(x):
    i = plsc.bitcast(x, jnp.int32)
    i = jnp.int32(0x5F3759DF) - (i >> 1)
    y = plsc.bitcast(i, jnp.float32)
    for _ in range(3):
        y = y * (jnp.float32(1.5) - jnp.float32(0.5) * x * y * y)
    return y


def _rsqrt_scalar(x):
    return jnp.max(_rsqrt16(jnp.broadcast_to(x, (16,))))


def _ffs(mask):
    lane = plsc.all_reduce_ffs(mask)
    if lane.ndim:
        lane = jnp.max(lane)
    return lane


_mesh = plsc.VectorSubcoreMesh(core_axis_name="c", subcore_axis_name="s")


@functools.partial(
    pl.kernel,
    out_type=(
        jax.ShapeDtypeStruct((LTM_SLOTS, VECTOR_DIM), jnp.float32),
        jax.ShapeDtypeStruct((PAD_S,), jnp.float32),
    ),
    mesh=_mesh,
    scratch_types=[
        pltpu.VMEM((VECTOR_DIM,), jnp.float32),
        pltpu.VMEM((16,), jnp.float32),
        pltpu.VMEM((LTM_SLOTS, VECTOR_DIM), jnp.float32),
        pltpu.VMEM((PAD_S,), jnp.float32),
    ],
    compiler_params=pltpu.CompilerParams(needs_layout_passes=False),
)
def _sc_kernel(iv_hbm, par_hbm, ltm_hbm, str_hbm, outm_hbm, outs_hbm,
               v_v, par_v, ltm_v, str_v):
    is_w0 = jnp.logical_and(lax.axis_index("c") == 0, lax.axis_index("s") == 0)

    @pl.when(is_w0)
    def _():
        pltpu.sync_copy(iv_hbm, v_v)
        pltpu.sync_copy(par_hbm, par_v)
        pltpu.sync_copy(ltm_hbm, ltm_v)
        pltpu.sync_copy(str_hbm, str_v)

        imp = jnp.max(par_v[...])

        v = [v_v[pl.ds(16 * j, 16)] for j in range(4)]
        nsv = jnp.sum(v[0] * v[0] + v[1] * v[1] + v[2] * v[2] + v[3] * v[3])
        inv1 = jnp.minimum(_rsqrt_scalar(nsv), jnp.float32(1e12))
        v1 = [vj * inv1 for vj in v]
        nsv1 = nsv * inv1 * inv1
        inv2 = jnp.minimum(_rsqrt_scalar(nsv1), jnp.float32(1e12))
        vn = [vj * inv2 for vj in v1]

        lid = lax.iota(jnp.int32, 16)

        def row_body(i, carry):
            best_q, best_i, max_ns = carry
            r = [ltm_v[i, pl.ds(16 * j, 16)] for j in range(4)]
            dacc = r[0] * vn[0] + r[1] * vn[1] + r[2] * vn[2] + r[3] * vn[3]
            nacc = r[0] * r[0] + r[1] * r[1] + r[2] * r[2] + r[3] * r[3]
            d = jnp.sum(dacc)
            ns = jnp.sum(nacc)
            q = d * jnp.minimum(_rsqrt_scalar(ns), jnp.float32(1e8))
            better = q > best_q
            best_i = jnp.where(better, i, best_i)
            best_q = jnp.maximum(best_q, q)
            max_ns = jnp.maximum(max_ns, ns)
            return best_q, best_i, max_ns

        best_q, best_i, max_ns = lax.fori_loop(
            0, LTM_SLOTS, row_body,
            (jnp.float32(NEG_BIG), jnp.int32(0), jnp.float32(0.0)))

        best_s = jnp.float32(3.4e38)
        weak_i = jnp.int32(0)
        for k in range(NCHUNK):
            sk = str_v[pl.ds(16 * k, 16)]
            cmin = jnp.min(sk)
            lane = _ffs(sk == cmin)
            better = cmin < best_s
            weak_i = jnp.where(better, 16 * k + lane, weak_i)
            best_s = jnp.minimum(best_s, cmin)

        all_empty = max_ns < jnp.float32(1e-12)
        reinforce = jnp.logical_and(jnp.logical_not(all_empty),
                                    best_q > jnp.float32(SIMILARITY_THRESHOLD))
        slot = jnp.where(reinforce, best_i, weak_i)
        store_b = imp > jnp.float32(IMPORTANCE_THRESHOLD)

        msi_idx = jnp.broadcast_to(best_i, (16,))
        old = [plsc.load_gather(ltm_v, [msi_idx, lid + 16 * j])
               for j in range(4)]
        str_msi = jnp.max(plsc.load_gather(str_v, [msi_idx]))
        boosted = jnp.minimum(str_msi + imp * jnp.float32(BOOST_FACTOR),
                              jnp.float32(1.0))
        new_str = jnp.where(reinforce, boosted, imp)

        merged = [jnp.float32(OLD_WEIGHT) * old[j]
                  + jnp.float32(NEW_WEIGHT) * v1[j] for j in range(4)]
        mns = jnp.sum(merged[0] * merged[0] + merged[1] * merged[1]
                      + merged[2] * merged[2] + merged[3] * merged[3])
        invm = jnp.minimum(_rsqrt_scalar(mns), jnp.float32(1e12))
        slot_vec = [jnp.where(reinforce, merged[j] * invm, v1[j])
                    for j in range(4)]

        @pl.when(store_b)
        def _write():
            sidx = jnp.broadcast_to(slot, (16,))
            for j in range(4):
                plsc.store_scatter(ltm_v, [sidx, lid + 16 * j], slot_vec[j])
            plsc.store_scatter(str_v, [sidx],
                               jnp.broadcast_to(new_str, (16,)))

        for k in range(NCHUNK):
            x = str_v[pl.ds(16 * k, 16)] * jnp.float32(DECAY_RATE)
            x = x * (x > jnp.float32(0.01)).astype(jnp.float32)
            str_v[pl.ds(16 * k, 16)] = x

        pltpu.sync_copy(ltm_v, outm_hbm)
        pltpu.sync_copy(str_v, outs_hbm)


def kernel(input_vector, importance_score, ltm_matrix, ltm_strengths):
    par = jnp.full((16,), importance_score, dtype=jnp.float32)
    str_p = jnp.concatenate(
        [ltm_strengths,
         jnp.full((PAD_S - LTM_SLOTS,), PAD_STRENGTH, dtype=jnp.float32)])
    outm, outs = _sc_kernel(input_vector, par, ltm_matrix, str_p)
    return outm, outs[:LTM_SLOTS]
